# baseline (speedup 1.0000x reference)
import jax
import jax.numpy as jnp
from jax.experimental import pallas as pl

_N, _D = 100000, 256
_BR = 1000


def _body(x_ref, a_ref, h_ref, w_ref, o_ref):
    beta = jnp.clip(1.0 - w_ref[...] * a_ref[...], 0.0, 1.0)
    o_ref[...] = h_ref[...] * beta + x_ref[...]


def kernel(x, agg_n, his_x, inv_w):
    a2 = agg_n.reshape(_N, 1)
    w2 = inv_w.reshape(_N, 1)
    return pl.pallas_call(
        _body,
        grid=(_N // _BR,),
        in_specs=[
            pl.BlockSpec((_BR, _D), lambda i: (i, 0)),
            pl.BlockSpec((_BR, 1), lambda i: (i, 0)),
            pl.BlockSpec((_BR, _D), lambda i: (i, 0)),
            pl.BlockSpec((_BR, 1), lambda i: (i, 0)),
        ],
        out_specs=pl.BlockSpec((_BR, _D), lambda i: (i, 0)),
        out_shape=jax.ShapeDtypeStruct((_N, _D), jnp.float32),
    )(x, a2, his_x, w2)


# speedup vs baseline: 1.6979x; 1.6979x over previous
import jax
import jax.numpy as jnp
from jax.experimental import pallas as pl

_N, _D = 100000, 256
_BR = 1000


def _body(x_ref, a_ref, h_ref, w_ref, o_ref):
    beta = jnp.clip(1.0 - w_ref[0] * a_ref[0], 0.0, 1.0)
    beta = beta.reshape(_BR, 1)
    o_ref[...] = h_ref[...] * beta + x_ref[...]


def kernel(x, agg_n, his_x, inv_w):
    a2 = agg_n.reshape(_N // _BR, 1, _BR)
    w2 = inv_w.reshape(_N // _BR, 1, _BR)
    return pl.pallas_call(
        _body,
        grid=(_N // _BR,),
        in_specs=[
            pl.BlockSpec((_BR, _D), lambda i: (i, 0)),
            pl.BlockSpec((1, 1, _BR), lambda i: (i, 0, 0)),
            pl.BlockSpec((_BR, _D), lambda i: (i, 0)),
            pl.BlockSpec((1, 1, _BR), lambda i: (i, 0, 0)),
        ],
        out_specs=pl.BlockSpec((_BR, _D), lambda i: (i, 0)),
        out_shape=jax.ShapeDtypeStruct((_N, _D), jnp.float32),
    )(x, a2, his_x, w2)


# speedup vs baseline: 2.0683x; 1.2182x over previous
import jax
import jax.numpy as jnp
from jax.experimental import pallas as pl

_N, _D = 100000, 256
_BR = 2000


def _body(x_ref, a_ref, h_ref, w_ref, o_ref):
    beta = jnp.clip(1.0 - w_ref[0] * a_ref[0], 0.0, 1.0)
    beta = beta.reshape(_BR, 1)
    o_ref[...] = h_ref[...] * beta + x_ref[...]


def kernel(x, agg_n, his_x, inv_w):
    a2 = agg_n.reshape(_N // _BR, 1, _BR)
    w2 = inv_w.reshape(_N // _BR, 1, _BR)
    return pl.pallas_call(
        _body,
        grid=(_N // _BR,),
        in_specs=[
            pl.BlockSpec((_BR, _D), lambda i: (i, 0)),
            pl.BlockSpec((1, 1, _BR), lambda i: (i, 0, 0)),
            pl.BlockSpec((_BR, _D), lambda i: (i, 0)),
            pl.BlockSpec((1, 1, _BR), lambda i: (i, 0, 0)),
        ],
        out_specs=pl.BlockSpec((_BR, _D), lambda i: (i, 0)),
        out_shape=jax.ShapeDtypeStruct((_N, _D), jnp.float32),
    )(x, a2, his_x, w2)


# speedup vs baseline: 2.1486x; 1.0388x over previous
import jax
import jax.numpy as jnp
from jax.experimental import pallas as pl

_N, _D = 100000, 256
_BR = 4000


def _body(x_ref, a_ref, h_ref, w_ref, o_ref):
    beta = jnp.clip(1.0 - w_ref[0] * a_ref[0], 0.0, 1.0)
    beta = beta.reshape(_BR, 1)
    o_ref[...] = h_ref[...] * beta + x_ref[...]


def kernel(x, agg_n, his_x, inv_w):
    a2 = agg_n.reshape(_N // _BR, 1, _BR)
    w2 = inv_w.reshape(_N // _BR, 1, _BR)
    return pl.pallas_call(
        _body,
        grid=(_N // _BR,),
        in_specs=[
            pl.BlockSpec((_BR, _D), lambda i: (i, 0)),
            pl.BlockSpec((1, 1, _BR), lambda i: (i, 0, 0)),
            pl.BlockSpec((_BR, _D), lambda i: (i, 0)),
            pl.BlockSpec((1, 1, _BR), lambda i: (i, 0, 0)),
        ],
        out_specs=pl.BlockSpec((_BR, _D), lambda i: (i, 0)),
        out_shape=jax.ShapeDtypeStruct((_N, _D), jnp.float32),
    )(x, a2, his_x, w2)
